# baseline (speedup 1.0000x reference)
import jax
import jax.numpy as jnp
from jax import lax
from jax.experimental import pallas as pl
from jax.experimental.pallas import tpu as pltpu
from jax.experimental.pallas import tpu_sc as plsc

B, S, D = 4, 4096, 1024
H = S // 2
CC = 256
NBUF = 2
NCHC = H // CC
NCH = 2 * NCHC
NG = NCH // NBUF


def _shuffle_body(mem_in, out, buf0, buf1, sin0, sin1, sout0, sout1):
    mem_hbm = mem_in.reshape(B, 2, H, D)
    cid = lax.axis_index("c")

    bufs = (buf0, buf1)
    sin = (sin0, sin1)
    sout = (sout0, sout1)

    def bk(c):
        return 2 * cid + c // NCHC, (c % NCHC) * CC

    def start_in(j, c):
        bb, k = bk(c)
        pltpu.async_copy(mem_hbm.at[bb, 0, pl.ds(k, CC), :],
                         bufs[j].at[:, 0, :], sin[j])
        pltpu.async_copy(mem_hbm.at[bb, 1, pl.ds(k, CC), :],
                         bufs[j].at[:, 1, :], sin[j])

    def wait_in(j):
        pltpu.make_async_copy(mem_hbm.at[0, 0, pl.ds(0, CC), :],
                              bufs[j].at[:, 0, :], sin[j]).wait()
        pltpu.make_async_copy(mem_hbm.at[0, 1, pl.ds(0, CC), :],
                              bufs[j].at[:, 1, :], sin[j]).wait()

    def start_out(j, c):
        bb, k = bk(c)
        pltpu.async_copy(bufs[j].reshape(2 * CC, D),
                         out.at[bb, pl.ds(2 * k, 2 * CC), :], sout[j])

    def wait_out(j):
        pltpu.make_async_copy(bufs[j].reshape(2 * CC, D),
                              out.at[0, pl.ds(0, 2 * CC), :], sout[j]).wait()

    for j in range(NBUF - 1):
        start_in(j, j)

    def ring_round(i, carry):
        for jj in range(NBUF):
            c = i * NBUF + jj
            jw = (jj + NBUF - 1) % NBUF

            @pl.when(c >= 1)
            def _():
                wait_out(jw)

            @pl.when(c + NBUF - 1 < NCH)
            def _():
                start_in(jw, c + NBUF - 1)

            wait_in(jj)
            start_out(jj, c)
        return carry

    lax.fori_loop(0, NG, ring_round, 0)
    wait_out((NCH - 1) % NBUF)


def kernel(mem):
    return pl.kernel(
        _shuffle_body,
        out_type=jax.ShapeDtypeStruct((B, S, D), jnp.float32),
        mesh=plsc.ScalarSubcoreMesh(axis_name="c", num_cores=2),
        scratch_types=(
            [pltpu.VMEM_SHARED((CC, 2, D), jnp.float32)] * NBUF
            + [pltpu.SemaphoreType.DMA] * (2 * NBUF)
        ),
    )(mem)
